# Initial kernel scaffold; baseline (speedup 1.0000x reference)
import jax
import jax.numpy as jnp
from jax.experimental import pallas as pl


def kernel(x, table, W1, b1, W2, b2):
    raise NotImplementedError("write your pallas kernel here")


# speedup vs baseline: 1.0541x; 1.0541x over previous
import functools

import jax
import jax.numpy as jnp
from jax import lax
from jax.experimental import pallas as pl
from jax.experimental.pallas import tpu as pltpu
from jax.experimental.pallas import tpu_sc as plsc

B = 4096
SEQ = 200
DIM = 64

NC = 2
NS = 16
NW = NC * NS
B_PER_W = B // NW
HALF = SEQ // 2


def _sc_pool_call(x3, table):
    mesh = plsc.VectorSubcoreMesh(
        core_axis_name="c", subcore_axis_name="s", num_cores=NC,
        num_subcores=NS)

    @functools.partial(
        pl.kernel,
        out_type=jax.ShapeDtypeStruct((B, DIM), jnp.float32),
        mesh=mesh,
        compiler_params=pltpu.CompilerParams(use_tc_tiling_on_sc=False),
        scratch_types=[
            pltpu.VMEM((2, HALF), jnp.int32),
            pltpu.VMEM((HALF, DIM), jnp.float32),
            pltpu.VMEM((HALF, DIM), jnp.float32),
            pltpu.VMEM((B_PER_W, DIM), jnp.float32),
            pltpu.SemaphoreType.DMA,
            pltpu.SemaphoreType.DMA,
        ],
    )
    def sc_pool(x_hbm, table_hbm, out_hbm, idx_v, rows0_v, rows1_v,
                pool_v, sem0, sem1):
        wid = lax.axis_index("s") * NC + lax.axis_index("c")
        base = wid * B_PER_W

        def sample_body(i, _):
            pltpu.sync_copy(x_hbm.at[base + i], idx_v)
            cp0 = pltpu.async_copy(table_hbm.at[idx_v.at[0]], rows0_v, sem0)
            cp1 = pltpu.async_copy(table_hbm.at[idx_v.at[1]], rows1_v, sem1)
            cp0.wait()
            cp1.wait()

            def acc_body(r, accs):
                a0, a1, a2, a3 = accs
                a0 = a0 + rows0_v[r, pl.ds(0, 16)] + rows1_v[r, pl.ds(0, 16)]
                a1 = a1 + rows0_v[r, pl.ds(16, 16)] + rows1_v[r, pl.ds(16, 16)]
                a2 = a2 + rows0_v[r, pl.ds(32, 16)] + rows1_v[r, pl.ds(32, 16)]
                a3 = a3 + rows0_v[r, pl.ds(48, 16)] + rows1_v[r, pl.ds(48, 16)]
                return (a0, a1, a2, a3)

            zero = jnp.zeros((16,), jnp.float32)
            a0, a1, a2, a3 = lax.fori_loop(
                0, HALF, acc_body, (zero, zero, zero, zero))
            scale = jnp.float32(1.0 / SEQ)
            pool_v[i, pl.ds(0, 16)] = a0 * scale
            pool_v[i, pl.ds(16, 16)] = a1 * scale
            pool_v[i, pl.ds(32, 16)] = a2 * scale
            pool_v[i, pl.ds(48, 16)] = a3 * scale
            return 0

        lax.fori_loop(0, B_PER_W, sample_body, 0)
        pltpu.sync_copy(pool_v, out_hbm.at[pl.ds(base, B_PER_W)])

    return sc_pool(x3, table)


def _mlp_body(m_ref, w1_ref, b1_ref, w2_ref, b2_ref, o_ref):
    h = jnp.dot(m_ref[...], w1_ref[...],
                preferred_element_type=jnp.float32) + b1_ref[...]
    h = jax.nn.sigmoid(jnp.maximum(h, 0.0))
    o = jnp.dot(h, w2_ref[...],
                preferred_element_type=jnp.float32) + b2_ref[...]
    o_ref[...] = jax.nn.sigmoid(o)


def _mlp_call(pooled, W1, b1, W2, b2):
    return pl.pallas_call(
        _mlp_body,
        out_shape=jax.ShapeDtypeStruct((B, 1), jnp.float32),
    )(pooled, W1, b1.reshape(1, 16), W2, b2.reshape(1, 1))


def kernel(x, table, W1, b1, W2, b2):
    x3 = x.astype(jnp.int32).reshape(B, 2, HALF)
    pooled = _sc_pool_call(x3, table)
    return _mlp_call(pooled, W1, b1, W2, b2)


# speedup vs baseline: 1.5809x; 1.4997x over previous
import functools

import jax
import jax.numpy as jnp
from jax import lax
from jax.experimental import pallas as pl
from jax.experimental.pallas import tpu as pltpu
from jax.experimental.pallas import tpu_sc as plsc

B = 4096
SEQ = 200
DIM = 64

NC = 2
NS = 16
NW = NC * NS
B_PER_W = B // NW
HALF = SEQ // 2
RING = 4


def _sc_pool_call(x3, table):
    mesh = plsc.VectorSubcoreMesh(
        core_axis_name="c", subcore_axis_name="s", num_cores=NC,
        num_subcores=NS)

    @functools.partial(
        pl.kernel,
        out_type=jax.ShapeDtypeStruct((B, DIM), jnp.float32),
        mesh=mesh,
        compiler_params=pltpu.CompilerParams(use_tc_tiling_on_sc=False),
        scratch_types=[
            pltpu.VMEM((B_PER_W, 2, HALF), jnp.int32),
            *[pltpu.VMEM((HALF, DIM), jnp.float32) for _ in range(RING)],
            pltpu.VMEM((B_PER_W, DIM), jnp.float32),
            *[pltpu.SemaphoreType.DMA for _ in range(RING)],
        ],
    )
    def sc_pool(x_hbm, table_hbm, out_hbm, idx_v, r0, r1, r2, r3,
                pool_v, s0, s1, s2, s3):
        rows = (r0, r1, r2, r3)
        sems = (s0, s1, s2, s3)
        wid = lax.axis_index("s") * NC + lax.axis_index("c")
        base = wid * B_PER_W
        pltpu.sync_copy(x_hbm.at[pl.ds(base, B_PER_W)], idx_v)

        for b in range(RING):
            pltpu.async_copy(
                table_hbm.at[idx_v.at[b // 2, b % 2]], rows[b], sems[b])

        scale = jnp.float32(1.0 / SEQ)
        zero = jnp.zeros((16,), jnp.float32)
        samples_per_group = RING // 2

        def outer(t, _):
            for k in range(samples_per_group):
                s = samples_per_group * t + k
                accs = (zero, zero, zero, zero)
                for hb in range(2):
                    b = 2 * k + hb
                    pltpu.make_async_copy(
                        table_hbm.at[pl.ds(0, HALF)], rows[b], sems[b]).wait()

                    rbuf = rows[b]

                    def acc_body(r, a, rbuf=rbuf):
                        return (
                            a[0] + rbuf[r, pl.ds(0, 16)],
                            a[1] + rbuf[r, pl.ds(16, 16)],
                            a[2] + rbuf[r, pl.ds(32, 16)],
                            a[3] + rbuf[r, pl.ds(48, 16)],
                        )

                    accs = lax.fori_loop(0, HALF, acc_body, accs, unroll=10)

                    s_next = s + samples_per_group

                    @pl.when(s_next < B_PER_W)
                    def _(b=b, hb=hb, s_next=s_next):
                        pltpu.async_copy(
                            table_hbm.at[idx_v.at[s_next, hb]],
                            rows[b], sems[b])

                pool_v[s, pl.ds(0, 16)] = accs[0] * scale
                pool_v[s, pl.ds(16, 16)] = accs[1] * scale
                pool_v[s, pl.ds(32, 16)] = accs[2] * scale
                pool_v[s, pl.ds(48, 16)] = accs[3] * scale
            return 0

        lax.fori_loop(0, B_PER_W // samples_per_group, outer, 0)
        pltpu.sync_copy(pool_v, out_hbm.at[pl.ds(base, B_PER_W)])

    return sc_pool(x3, table)


def _mlp_body(m_ref, w1_ref, b1_ref, w2_ref, b2_ref, o_ref):
    h = jnp.dot(m_ref[...], w1_ref[...],
                preferred_element_type=jnp.float32) + b1_ref[...]
    h = jax.nn.sigmoid(jnp.maximum(h, 0.0))
    o = jnp.dot(h, w2_ref[...],
                preferred_element_type=jnp.float32) + b2_ref[...]
    o_ref[...] = jax.nn.sigmoid(o)


def _mlp_call(pooled, W1, b1, W2, b2):
    return pl.pallas_call(
        _mlp_body,
        out_shape=jax.ShapeDtypeStruct((B, 1), jnp.float32),
    )(pooled, W1, b1.reshape(1, 16), W2, b2.reshape(1, 1))


def kernel(x, table, W1, b1, W2, b2):
    x3 = x.astype(jnp.int32).reshape(B, 2, HALF)
    pooled = _sc_pool_call(x3, table)
    return _mlp_call(pooled, W1, b1, W2, b2)


# speedup vs baseline: 1.6096x; 1.0182x over previous
import functools

import jax
import jax.numpy as jnp
from jax import lax
from jax.experimental import pallas as pl
from jax.experimental.pallas import tpu as pltpu
from jax.experimental.pallas import tpu_sc as plsc

B = 4096
SEQ = 200
DIM = 64

NC = 2
NS = 16
NW = NC * NS
B_PER_W = B // NW
CH0 = 104
CH1 = 96
RING = 4


def _sc_pool_call(x2, table):
    mesh = plsc.VectorSubcoreMesh(
        core_axis_name="c", subcore_axis_name="s", num_cores=NC,
        num_subcores=NS)

    @functools.partial(
        pl.kernel,
        out_type=jax.ShapeDtypeStruct((B, DIM), jnp.float32),
        mesh=mesh,
        compiler_params=pltpu.CompilerParams(use_tc_tiling_on_sc=False),
        scratch_types=[
            pltpu.VMEM((B_PER_W, SEQ), jnp.int32),
            *[pltpu.VMEM(((CH0, CH1)[b % 2], DIM), jnp.float32)
              for b in range(RING)],
            pltpu.VMEM((B_PER_W, DIM), jnp.float32),
            *[pltpu.SemaphoreType.DMA for _ in range(RING)],
        ],
    )
    def sc_pool(x_hbm, table_hbm, out_hbm, idx_v, r0, r1, r2, r3,
                pool_v, s0, s1, s2, s3):
        rows = (r0, r1, r2, r3)
        sems = (s0, s1, s2, s3)
        chlen = (CH0, CH1)
        choff = (0, CH0)
        wid = lax.axis_index("s") * NC + lax.axis_index("c")
        base = wid * B_PER_W
        pltpu.sync_copy(x_hbm.at[pl.ds(base, B_PER_W)], idx_v)

        for b in range(RING):
            pltpu.async_copy(
                table_hbm.at[idx_v.at[b // 2, pl.ds(choff[b % 2],
                                                    chlen[b % 2])]],
                rows[b], sems[b])

        scale = jnp.float32(1.0 / SEQ)
        zero = jnp.zeros((16,), jnp.float32)
        samples_per_group = RING // 2

        def outer(t, _):
            for k in range(samples_per_group):
                s = samples_per_group * t + k
                accs = (zero, zero, zero, zero)
                for hb in range(2):
                    b = 2 * k + hb
                    pltpu.make_async_copy(
                        table_hbm.at[pl.ds(0, chlen[hb])],
                        rows[b], sems[b]).wait()

                    rbuf = rows[b]

                    def acc_body(r, a, rbuf=rbuf):
                        return (
                            a[0] + rbuf[r, pl.ds(0, 16)],
                            a[1] + rbuf[r, pl.ds(16, 16)],
                            a[2] + rbuf[r, pl.ds(32, 16)],
                            a[3] + rbuf[r, pl.ds(48, 16)],
                        )

                    accs = lax.fori_loop(0, chlen[hb], acc_body, accs,
                                         unroll=8)

                    s_next = s + samples_per_group

                    @pl.when(s_next < B_PER_W)
                    def _(b=b, hb=hb, s_next=s_next):
                        pltpu.async_copy(
                            table_hbm.at[idx_v.at[s_next,
                                                  pl.ds(choff[hb],
                                                        chlen[hb])]],
                            rows[b], sems[b])

                pool_v[s, pl.ds(0, 16)] = accs[0] * scale
                pool_v[s, pl.ds(16, 16)] = accs[1] * scale
                pool_v[s, pl.ds(32, 16)] = accs[2] * scale
                pool_v[s, pl.ds(48, 16)] = accs[3] * scale
            return 0

        lax.fori_loop(0, B_PER_W // samples_per_group, outer, 0)
        pltpu.sync_copy(pool_v, out_hbm.at[pl.ds(base, B_PER_W)])

    return sc_pool(x2, table)


def _mlp_body(m_ref, w1_ref, b1_ref, w2_ref, b2_ref, o_ref):
    h = jnp.dot(m_ref[...], w1_ref[...],
                preferred_element_type=jnp.float32) + b1_ref[...]
    h = jax.nn.sigmoid(jnp.maximum(h, 0.0))
    o = jnp.dot(h, w2_ref[...],
                preferred_element_type=jnp.float32) + b2_ref[...]
    o_ref[...] = jax.nn.sigmoid(o)


def _mlp_call(pooled, W1, b1, W2, b2):
    return pl.pallas_call(
        _mlp_body,
        out_shape=jax.ShapeDtypeStruct((B, 1), jnp.float32),
    )(pooled, W1, b1.reshape(1, 16), W2, b2.reshape(1, 1))


def kernel(x, table, W1, b1, W2, b2):
    pooled = _sc_pool_call(x.astype(jnp.int32), table)
    return _mlp_call(pooled, W1, b1, W2, b2)
